# baseline (speedup 1.0000x reference)
import functools

import jax
import jax.numpy as jnp
import numpy as np
from jax import lax
from jax.experimental import pallas as pl
from jax.experimental.pallas import tpu as pltpu
from jax.experimental.pallas import tpu_sc as plsc

_N = 1000
_NW = 32
_BPW = 32
_C = 256
_S = 49
_SA = 32
_SB = 24
_SPAD = 56


def _body(bxl, lvl, gy, gx, f2, f3, f4, f5, out,
          bx_v, lvl_v, gy_v, gx_v, idx_a, idx_b, wt_a, wt_b,
          crn_a, crn_b, out_v, out_w, sem_a, sem_b, sem_o):
    wid = lax.axis_index("c") * 16 + lax.axis_index("s")
    base = wid * _BPW

    pltpu.sync_copy(bxl.at[pl.ds(base * 4, _BPW * 4)], bx_v.at[pl.ds(0, _BPW * 4)])
    pltpu.sync_copy(lvl.at[pl.ds(base, _BPW)], lvl_v.at[pl.ds(0, _BPW)])
    pltpu.sync_copy(gy, gy_v)
    pltpu.sync_copy(gx, gx_v)

    feats = (f2, f3, f4, f5)

    def build_and_fire(j, groups, idx_r, wt_r, crn_r, sem):
        bv = bx_v[pl.ds(j * 4, 16)]
        y1s = bv[0]
        x1s = bv[1]
        y2s = bv[2]
        x2s = bv[3]
        lvl_s = lvl_v[pl.ds(j, 16)][0]
        wi = jnp.int32(256) >> lvl_s
        wm1i = wi - 1
        wm1f = wm1i.astype(jnp.float32)
        dys = y2s - y1s
        dxs = x2s - x1s

        span = groups[0][2]
        for gbase, ioff, _sp in groups:
            gyf = gy_v[pl.ds(gbase, 16)]
            gxf = gx_v[pl.ds(gbase, 16)]
            ys = y1s * wm1f + (gyf * dys) * wm1f
            xs = x1s * wm1f + (gxf * dxs) * wm1f
            y0 = ys.astype(jnp.int32)
            x0 = xs.astype(jnp.int32)
            wy = ys - y0.astype(jnp.float32)
            wx = xs - x0.astype(jnp.float32)
            y0c = jnp.minimum(y0, wm1i)
            x0c = jnp.minimum(x0, wm1i)
            y1c = jnp.minimum(y0c + 1, wm1i)
            x1c = jnp.minimum(x0c + 1, wm1i)
            cy = 1.0 - wy
            cx = 1.0 - wx
            yb0 = y0c * wi
            yb1 = y1c * wi
            idx_r[0, pl.ds(0 * span + ioff, 16)] = yb0 + x0c
            idx_r[0, pl.ds(1 * span + ioff, 16)] = yb0 + x1c
            idx_r[0, pl.ds(2 * span + ioff, 16)] = yb1 + x0c
            idx_r[0, pl.ds(3 * span + ioff, 16)] = yb1 + x1c
            wt_r[pl.ds(0 * _SPAD + gbase, 16)] = cy * cx
            wt_r[pl.ds(1 * _SPAD + gbase, 16)] = cy * wx
            wt_r[pl.ds(2 * _SPAD + gbase, 16)] = wy * cx
            wt_r[pl.ds(3 * _SPAD + gbase, 16)] = wy * wx

        for lev in range(4):
            @pl.when(lvl_s == lev)
            def _():
                pltpu.async_copy(feats[lev].at[idx_r.at[0]], crn_r, sem)

    ga = ((0, 0, _SA), (16, 16, _SA))
    gb = ((32, 0, _SB), (40, 8, _SB))

    def wait_half(idx_r, crn_r, sem):
        pltpu.make_async_copy(f2.at[idx_r.at[0]], crn_r, sem).wait()

    def compute_half(s_lo, s_hi, span, wt_r, crn_r, ov):
        def s_body(s, c2):
            r = s - s_lo
            gyi = s // 7
            gxi = s - gyi * 7
            wtl = wt_r[pl.ds(0 * _SPAD + s, 16)][0]
            wtr = wt_r[pl.ds(1 * _SPAD + s, 16)][0]
            wbl = wt_r[pl.ds(2 * _SPAD + s, 16)][0]
            wbr = wt_r[pl.ds(3 * _SPAD + s, 16)][0]
            for c in range(_C // 16):
                cl = pl.ds(c * 16, 16)
                ov[gyi, gxi, cl] = (crn_r[0 * span + r, cl] * wtl
                             + crn_r[1 * span + r, cl] * wtr
                             + crn_r[2 * span + r, cl] * wbl
                             + crn_r[3 * span + r, cl] * wbr)
            return c2

        lax.fori_loop(s_lo, s_hi, s_body, 0)

    nb = jnp.minimum(_BPW, _N - base)
    build_and_fire(jnp.int32(0), ga, idx_a, wt_a, crn_a, sem_a)

    def one_box(j, jn, ov):
        wait_half(idx_a, crn_a, sem_a)
        build_and_fire(j, gb, idx_b, wt_b, crn_b, sem_b)
        compute_half(0, _SA, _SA, wt_a, crn_a, ov)
        wait_half(idx_b, crn_b, sem_b)
        build_and_fire(jn, ga, idx_a, wt_a, crn_a, sem_a)
        compute_half(_SA, _S, _SB, wt_b, crn_b, ov)
        pltpu.async_copy(ov, out.at[base + j], sem_o)

    def pair_body(i, carry):
        j0 = 2 * i
        j1 = j0 + 1
        j2 = jnp.minimum(j0 + 2, nb - 1)

        @pl.when(i > 0)
        def _():
            pltpu.make_async_copy(out_v, out.at[base], sem_o).wait()
            pltpu.make_async_copy(out_w, out.at[base], sem_o).wait()

        one_box(j0, j1, out_v)
        one_box(j1, j2, out_w)
        return carry

    lax.fori_loop(0, nb // 2, pair_body, 0)
    wait_half(idx_a, crn_a, sem_a)
    pltpu.make_async_copy(out_v, out.at[base], sem_o).wait()
    pltpu.make_async_copy(out_w, out.at[base], sem_o).wait()


@functools.partial(jax.jit, static_argnums=())
def kernel(boxes, feat2, feat3, feat4, feat5):
    b = boxes[0]
    y1, x1, y2, x2 = b[:, 0], b[:, 1], b[:, 2], b[:, 3]
    h = y2 - y1
    w = x2 - x1
    image_area = 1024.0 * 1024.0
    roi_level = jnp.log(jnp.sqrt(h * w) / (224.0 / np.sqrt(image_area))) / np.log(2.0)
    lvl = jnp.clip(4 + jnp.round(roi_level).astype(jnp.int32), 2, 5) - 2

    bxl = jnp.zeros((4 * (_N + 24),), jnp.float32).at[:4 * _N].set(b.reshape(-1))
    lvl_pad = jnp.zeros((_N + 24,), jnp.int32).at[:_N].set(lvl)

    grid = jnp.arange(7, dtype=jnp.float32) / float(7 - 1)
    s_ids = np.minimum(np.arange(_SPAD), _S - 1)
    gyh = grid[s_ids // 7]
    gxh = grid[s_ids % 7]

    feats = [feat2[0].reshape(-1, _C), feat3[0].reshape(-1, _C),
             feat4[0].reshape(-1, _C), feat5[0].reshape(-1, _C)]

    mesh = plsc.VectorSubcoreMesh(core_axis_name="c", subcore_axis_name="s")
    out = pl.kernel(
        _body,
        out_type=jax.ShapeDtypeStruct((_N, 7, 7, _C), jnp.float32),
        mesh=mesh,
        scratch_types=[
            pltpu.VMEM((_BPW * 4 + 16,), jnp.float32),
            pltpu.VMEM((_BPW + 16,), jnp.int32),
            pltpu.VMEM((_SPAD,), jnp.float32),
            pltpu.VMEM((_SPAD,), jnp.float32),
            pltpu.VMEM((1, 4 * _SA), jnp.int32),
            pltpu.VMEM((1, 4 * _SB), jnp.int32),
            pltpu.VMEM((4 * _SPAD + 16,), jnp.float32),
            pltpu.VMEM((4 * _SPAD + 16,), jnp.float32),
            pltpu.VMEM((4 * _SA, _C), jnp.float32),
            pltpu.VMEM((4 * _SB, _C), jnp.float32),
            pltpu.VMEM((7, 7, _C), jnp.float32),
            pltpu.VMEM((7, 7, _C), jnp.float32),
            pltpu.SemaphoreType.DMA,
            pltpu.SemaphoreType.DMA,
            pltpu.SemaphoreType.DMA,
        ],
    )(bxl, lvl_pad, gyh, gxh, *feats)
    return out[None]


# speedup vs baseline: 1.9764x; 1.9764x over previous
import functools

import jax
import jax.numpy as jnp
import numpy as np
from jax import lax
from jax.experimental import pallas as pl
from jax.experimental.pallas import tpu as pltpu
from jax.experimental.pallas import tpu_sc as plsc

_N = 1000
_NW = 32
_BPW = 32
_C = 256
_S = 49
_SA = 32
_SB = 24
_SPAD = 56


def _body(bxl, lvl, gy, gx, f2, f3, f4, f5, out,
          bx_v, lvl_v, gy_v, gx_v, idx_a, idx_b, wt_a, wt_b,
          crn_a, crn_b, out_v, sem_a, sem_b):
    wid = lax.axis_index("c") * 16 + lax.axis_index("s")
    base = wid * _BPW

    pltpu.sync_copy(bxl.at[pl.ds(base * 4, _BPW * 4)], bx_v.at[pl.ds(0, _BPW * 4)])
    pltpu.sync_copy(lvl.at[pl.ds(base, _BPW)], lvl_v.at[pl.ds(0, _BPW)])
    pltpu.sync_copy(gy, gy_v)
    pltpu.sync_copy(gx, gx_v)

    feats = (f2, f3, f4, f5)

    def build_and_fire(j, groups, idx_r, wt_r, crn_r, sem):
        bv = bx_v[pl.ds(j * 4, 16)]
        y1s = bv[0]
        x1s = bv[1]
        y2s = bv[2]
        x2s = bv[3]
        lvl_s = lvl_v[pl.ds(j, 16)][0]
        wi = jnp.int32(256) >> lvl_s
        wm1i = wi - 1
        wm1f = wm1i.astype(jnp.float32)
        dys = y2s - y1s
        dxs = x2s - x1s

        for gbase, ioff in groups:
            gyf = gy_v[pl.ds(gbase, 16)]
            gxf = gx_v[pl.ds(gbase, 16)]
            ys = y1s * wm1f + (gyf * dys) * wm1f
            xs = x1s * wm1f + (gxf * dxs) * wm1f
            y0 = ys.astype(jnp.int32)
            x0 = xs.astype(jnp.int32)
            wy = ys - y0.astype(jnp.float32)
            wx = xs - x0.astype(jnp.float32)
            y0c = jnp.minimum(y0, wm1i)
            x0c = jnp.minimum(x0, wm1i)
            y1c = jnp.minimum(y0c + 1, wm1i)
            x1c = jnp.minimum(x0c + 1, wm1i)
            cy = 1.0 - wy
            cx = 1.0 - wx
            yb0 = y0c * wi
            yb1 = y1c * wi
            sl = pl.ds(ioff, 16)
            idx_r[0, sl] = yb0 + x0c
            idx_r[1, sl] = yb0 + x1c
            idx_r[2, sl] = yb1 + x0c
            idx_r[3, sl] = yb1 + x1c
            wt_r[pl.ds(0 * _SPAD + gbase, 16)] = cy * cx
            wt_r[pl.ds(1 * _SPAD + gbase, 16)] = cy * wx
            wt_r[pl.ds(2 * _SPAD + gbase, 16)] = wy * cx
            wt_r[pl.ds(3 * _SPAD + gbase, 16)] = wy * wx

        for lev in range(4):
            @pl.when(lvl_s == lev)
            def _():
                for k in range(4):
                    pltpu.async_copy(feats[lev].at[idx_r.at[k]],
                                     crn_r.at[k], sem)

    ga = ((0, 0), (16, 16))
    gb = ((32, 0), (40, 8))

    def wait_half(idx_r, crn_r, sem):
        for k in range(4):
            pltpu.make_async_copy(f2.at[idx_r.at[k]], crn_r.at[k], sem).wait()

    def compute_half(s_lo, s_hi, wt_r, crn_r):
        def s_body(s, c2):
            r = s - s_lo
            wtl = wt_r[pl.ds(0 * _SPAD + s, 16)][0]
            wtr = wt_r[pl.ds(1 * _SPAD + s, 16)][0]
            wbl = wt_r[pl.ds(2 * _SPAD + s, 16)][0]
            wbr = wt_r[pl.ds(3 * _SPAD + s, 16)][0]
            for c in range(_C // 16):
                cl = pl.ds(c * 16, 16)
                out_v[s, cl] = (crn_r[0, r, cl] * wtl + crn_r[1, r, cl] * wtr
                                + crn_r[2, r, cl] * wbl + crn_r[3, r, cl] * wbr)
            return c2

        lax.fori_loop(s_lo, s_hi, s_body, 0)

    nb = jnp.minimum(_BPW, _N - base)
    build_and_fire(jnp.int32(0), ga, idx_a, wt_a, crn_a, sem_a)

    def box_body(i, carry):
        jn = jnp.minimum(i + 1, nb - 1)
        wait_half(idx_a, crn_a, sem_a)
        build_and_fire(i, gb, idx_b, wt_b, crn_b, sem_b)
        compute_half(0, _SA, wt_a, crn_a)
        wait_half(idx_b, crn_b, sem_b)
        build_and_fire(jn, ga, idx_a, wt_a, crn_a, sem_a)
        compute_half(_SA, _S, wt_b, crn_b)
        pltpu.sync_copy(out_v, out.at[base + i])
        return carry

    lax.fori_loop(0, nb, box_body, 0)
    wait_half(idx_a, crn_a, sem_a)


@functools.partial(jax.jit, static_argnums=())
def kernel(boxes, feat2, feat3, feat4, feat5):
    b = boxes[0]
    y1, x1, y2, x2 = b[:, 0], b[:, 1], b[:, 2], b[:, 3]
    h = y2 - y1
    w = x2 - x1
    image_area = 1024.0 * 1024.0
    roi_level = jnp.log(jnp.sqrt(h * w) / (224.0 / np.sqrt(image_area))) / np.log(2.0)
    lvl = jnp.clip(4 + jnp.round(roi_level).astype(jnp.int32), 2, 5) - 2

    bxl = jnp.zeros((4 * (_N + 24),), jnp.float32).at[:4 * _N].set(b.reshape(-1))
    lvl_pad = jnp.zeros((_N + 24,), jnp.int32).at[:_N].set(lvl)

    grid = jnp.arange(7, dtype=jnp.float32) / float(7 - 1)
    s_ids = np.minimum(np.arange(_SPAD), _S - 1)
    gyh = grid[s_ids // 7]
    gxh = grid[s_ids % 7]

    feats = [feat2[0].reshape(-1, _C), feat3[0].reshape(-1, _C),
             feat4[0].reshape(-1, _C), feat5[0].reshape(-1, _C)]

    mesh = plsc.VectorSubcoreMesh(core_axis_name="c", subcore_axis_name="s")
    out = pl.kernel(
        _body,
        out_type=jax.ShapeDtypeStruct((_N, _S, _C), jnp.float32),
        mesh=mesh,
        scratch_types=[
            pltpu.VMEM((_BPW * 4 + 16,), jnp.float32),
            pltpu.VMEM((_BPW + 16,), jnp.int32),
            pltpu.VMEM((_SPAD,), jnp.float32),
            pltpu.VMEM((_SPAD,), jnp.float32),
            pltpu.VMEM((4, _SA), jnp.int32),
            pltpu.VMEM((4, _SB), jnp.int32),
            pltpu.VMEM((4 * _SPAD + 16,), jnp.float32),
            pltpu.VMEM((4 * _SPAD + 16,), jnp.float32),
            pltpu.VMEM((4, _SA, _C), jnp.float32),
            pltpu.VMEM((4, _SB, _C), jnp.float32),
            pltpu.VMEM((_S, _C), jnp.float32),
            pltpu.SemaphoreType.DMA,
            pltpu.SemaphoreType.DMA,
        ],
    )(bxl, lvl_pad, gyh, gxh, *feats)
    return out.reshape(1, _N, 7, 7, _C)
